# baseline (speedup 1.0000x reference)
import functools
import math

import jax
import jax.numpy as jnp
from jax import lax
from jax.experimental import pallas as pl
from jax.experimental.pallas import tpu as pltpu
from jax.experimental.pallas import tpu_sc as plsc

N = 10000
E = 320000
H = 128
NG = 50
NF = 128
NI = 6
CUT = 10.0
NGRAPH = 64
LOG2 = math.log(2.0)

NC = 2
NS = 16
HH = H // NC
EW = E // NS
CH = 125
NCHUNK = EW // CH
N_PAD = 10240
RPW = N_PAD // NS
RB = 64

TN = 2000
TE = 4000


def _ssp(x):
    return jnp.log(0.5 + 0.5 * jnp.exp(x))


def _embed_body(z_ref, emb_ref, lin1_ref, h_ref, x1_ref):
    zt = z_ref[...]
    ids = lax.broadcasted_iota(jnp.int32, (TN, 85), 1)
    oh = (zt == ids).astype(jnp.float32)
    h = jnp.dot(oh, emb_ref[...], preferred_element_type=jnp.float32)
    h_ref[...] = h
    x1 = jnp.dot(h, lin1_ref[...], preferred_element_type=jnp.float32)
    x1_ref[0] = x1[:, :HH]
    x1_ref[1] = x1[:, HH:]


def _embed(z2, emb, lin1):
    return pl.pallas_call(
        _embed_body,
        grid=(N // TN,),
        in_specs=[
            pl.BlockSpec((TN, 1), lambda i: (i, 0)),
            pl.BlockSpec((85, H), lambda i: (0, 0)),
            pl.BlockSpec((H, H), lambda i: (0, 0)),
        ],
        out_specs=[
            pl.BlockSpec((TN, H), lambda i: (i, 0)),
            pl.BlockSpec((NC, TN, HH), lambda i: (0, i, 0)),
        ],
        out_shape=[
            jax.ShapeDtypeStruct((N, H), jnp.float32),
            jax.ShapeDtypeStruct((NC, N, HH), jnp.float32),
        ],
    )(z2, emb, lin1)


def _filter_body(ew_ref, w1t_ref, b1_ref, w2t_ref, b2_ref, wf_ref):
    d = ew_ref[0]
    off = lax.broadcasted_iota(jnp.int32, (NG, 1), 0).astype(jnp.float32) * (
        CUT / (NG - 1))
    coeff = -0.5 / (CUT / (NG - 1)) ** 2
    diff = d - off
    ea_t = jnp.exp(coeff * diff * diff)
    hid_t = _ssp(jnp.dot(w1t_ref[...], ea_t, preferred_element_type=jnp.float32)
                 + b1_ref[...])
    wf_t = jnp.dot(w2t_ref[...], hid_t, preferred_element_type=jnp.float32) \
        + b2_ref[...]
    u2 = d * d * (jnp.pi / (2.0 * CUT)) ** 2
    cu = 1.0 + u2 * (-0.5 + u2 * (1.0 / 24 + u2 * (-1.0 / 720 + u2 * (
        1.0 / 40320 - u2 / 3628800))))
    wf_ref[...] = (wf_t * (cu * cu)).T


def _filter(ew3, w1t, b1c, w2t, b2c):
    full = lambda s: pl.BlockSpec(s, lambda i: (0, 0))
    return pl.pallas_call(
        _filter_body,
        grid=(E // TE,),
        in_specs=[
            pl.BlockSpec((1, 1, TE), lambda i: (i, 0, 0)),
            full((NF, NG)), full((NF, 1)), full((NF, NF)), full((NF, 1)),
        ],
        out_specs=pl.BlockSpec((TE, NF), lambda i: (i, 0)),
        out_shape=jax.ShapeDtypeStruct((E, NF), jnp.float32),
    )(ew3, w1t, b1c, w2t, b2c)


def _update_body(h_ref, a0_ref, a1_ref, w2c_ref, b2c_ref, lw_ref, lb_ref,
                 lin1n_ref, hn_ref, x1n_ref):
    agg = jnp.concatenate([a0_ref[0], a1_ref[0]], axis=1)
    t = _ssp(jnp.dot(agg, w2c_ref[...], preferred_element_type=jnp.float32)
             + b2c_ref[...])
    x2 = jnp.dot(t, lw_ref[...], preferred_element_type=jnp.float32) + lb_ref[...]
    hn = h_ref[...] + x2
    hn_ref[...] = hn
    x1n = jnp.dot(hn, lin1n_ref[...], preferred_element_type=jnp.float32)
    x1n_ref[0] = x1n[:, :HH]
    x1n_ref[1] = x1n[:, HH:]


def _update(h, aggp, w2c, b2c, lw, lb, lin1n):
    full = lambda s: pl.BlockSpec(s, lambda i: (0, 0))
    return pl.pallas_call(
        _update_body,
        grid=(N // TN,),
        in_specs=[
            pl.BlockSpec((TN, H), lambda i: (i, 0)),
            pl.BlockSpec((1, TN, HH), lambda i: (0, i, 0)),
            pl.BlockSpec((1, TN, HH), lambda i: (1, i, 0)),
            full((NF, H)), full((1, H)), full((H, H)), full((1, H)),
            full((H, H)),
        ],
        out_specs=[
            pl.BlockSpec((TN, H), lambda i: (i, 0)),
            pl.BlockSpec((NC, TN, HH), lambda i: (0, i, 0)),
        ],
        out_shape=[
            jax.ShapeDtypeStruct((N, H), jnp.float32),
            jax.ShapeDtypeStruct((NC, N, HH), jnp.float32),
        ],
    )(h, aggp, aggp, w2c, b2c, lw, lb, lin1n)


def _readout_body(h_ref, bat_ref, o1_ref, o1b_ref, o2_ref, o2b_ref, out_ref):
    i = pl.program_id(0)

    @pl.when(i == 0)
    def _():
        out_ref[...] = jnp.zeros_like(out_ref)

    h2 = _ssp(jnp.dot(h_ref[...], o1_ref[...], preferred_element_type=jnp.float32)
              + o1b_ref[...])
    y = jnp.sum(h2 * o2_ref[...], axis=1, keepdims=True) + o2b_ref[...]
    ids = lax.broadcasted_iota(jnp.int32, (TN, NGRAPH), 1)
    oh = (bat_ref[...] == ids).astype(jnp.float32)
    out_ref[...] += jnp.sum(oh * y, axis=0, keepdims=True)


def _readout(h, bat2, o1, o1b, o2row, o2b):
    full = lambda s: pl.BlockSpec(s, lambda i: (0, 0))
    return pl.pallas_call(
        _readout_body,
        grid=(N // TN,),
        in_specs=[
            pl.BlockSpec((TN, H), lambda i: (i, 0)),
            pl.BlockSpec((TN, 1), lambda i: (i, 0)),
            full((H, NGRAPH)), full((1, NGRAPH)), full((1, NGRAPH)),
            full((1, 1)),
        ],
        out_specs=pl.BlockSpec((1, NGRAPH), lambda i: (0, 0)),
        out_shape=jax.ShapeDtypeStruct((1, NGRAPH), jnp.float32),
    )(h, bat2, o1, o1b, o2row, o2b)


def _sc_body(x1_hbm, wf_hbm, src_hbm, dst_hbm, zer_hbm, out_hbm,
             srcv, dstv, rows4, wfv2, zbuf, aggsh,
             sg0, sg1, sw0, sw1, ss0, ss1, ss2):
    cid = lax.axis_index("c")
    sid = lax.axis_index("s")
    sg = (sg0, sg1)
    sw = (sw0, sw1)
    ss = (ss0, ss1, ss2)

    pltpu.sync_copy(zer_hbm, zbuf)
    for k in range(RPW // RB):
        pltpu.sync_copy(zbuf, aggsh.at[pl.ds(sid * RPW + k * RB, RB)])

    pltpu.sync_copy(src_hbm.at[sid], srcv)
    pltpu.sync_copy(dst_hbm.at[sid], dstv)
    plsc.subcore_barrier()

    ebase = sid * EW

    def issue(j, b, drain=True):
        if drain:
            pltpu.make_async_copy(rows4.at[b % 3], aggsh.at[dstv.at[j]],
                                  ss[b % 3]).wait()
        pltpu.async_copy(x1_hbm.at[cid].at[srcv.at[j]],
                         rows4.at[b % 3], sg[b % 2])
        pltpu.async_copy(
            wf_hbm.at[pl.ds(ebase + j * CH, CH), pl.ds(cid * HH, HH)],
            wfv2.at[b % 2], sw[b % 2])

    def process(j, b, issue_next, drain=True):
        pltpu.make_async_copy(x1_hbm.at[cid].at[srcv.at[j]],
                              rows4.at[b % 3], sg[b % 2]).wait()
        pltpu.make_async_copy(
            wf_hbm.at[pl.ds(ebase + j * CH, CH), pl.ds(cid * HH, HH)],
            wfv2.at[b % 2], sw[b % 2]).wait()

        rb = rows4.at[b % 3]
        wb = wfv2.at[b % 2]

        @plsc.parallel_loop(0, CH, 1, unroll=8)
        def _mul(e):
            for c in range(HH // 16):
                sl = pl.ds(c * 16, 16)
                rb[e, sl] = rb[e, sl] * wb[e, sl]

        pltpu.async_copy(rows4.at[b % 3], aggsh.at[dstv.at[j]], ss[b % 3],
                         add=True)
        if issue_next:
            issue(j + 2, b + 2, drain)

    issue(0, 0, drain=False)
    issue(1, 1, drain=False)

    process(0, 0, True, drain=False)
    for q in range(1, 6):
        process(q, q, True)

    def rnd(r, carry):
        j = 6 * r
        for q in range(6):
            process(j + q, q, True)
        return carry

    NMAIN = ((NCHUNK - 2) // 6) * 6
    lax.fori_loop(1, NMAIN // 6, rnd, 0)
    for q in range(NMAIN, NCHUNK):
        process(q, q, q + 2 < NCHUNK)
    for b in range(3):
        pltpu.make_async_copy(rows4.at[b], aggsh.at[dstv.at[0]], ss[b]).wait()
    plsc.subcore_barrier()

    for k in range(RPW // RB):
        sl = pl.ds(sid * RPW + k * RB, RB)
        pltpu.sync_copy(aggsh.at[sl], zbuf)
        pltpu.sync_copy(zbuf, out_hbm.at[cid].at[sl])


@functools.cache
def _build_sc_scatter():
    return pl.kernel(
        _sc_body,
        out_type=jax.ShapeDtypeStruct((NC, N_PAD, HH), jnp.float32),
        mesh=plsc.VectorSubcoreMesh(core_axis_name="c", subcore_axis_name="s"),
        scratch_types=[
            pltpu.VMEM((NCHUNK, CH), jnp.int32),
            pltpu.VMEM((NCHUNK, CH), jnp.int32),
            pltpu.VMEM((3, CH, HH), jnp.float32),
            pltpu.VMEM((2, CH, HH), jnp.float32),
            pltpu.VMEM((RB, HH), jnp.float32),
            pltpu.VMEM_SHARED((N_PAD, HH), jnp.float32),
        ] + [pltpu.SemaphoreType.DMA] * 7,
        compiler_params=pltpu.CompilerParams(use_tc_tiling_on_sc=False),
    )


def _sc_scatter(x1, wf4, src3, dst3, zer):
    return _build_sc_scatter()(x1, wf4, src3, dst3, zer)


def kernel(z, edge_index, edge_weight, batch, emb, mlp_w1, mlp_b1, mlp_w2,
           mlp_b2, conv_lin1, conv_lin2_w, conv_lin2_b, lin_w, lin_b,
           out1_w, out1_b, out2_w, out2_b):
    z2 = z.reshape(N, 1).astype(jnp.int32)
    bat2 = batch.reshape(N, 1).astype(jnp.int32)
    ew3 = edge_weight.reshape(E // TE, 1, TE)
    src3 = edge_index[0].reshape(NS, NCHUNK, CH).astype(jnp.int32)
    dst3 = edge_index[1].reshape(NS, NCHUNK, CH).astype(jnp.int32)
    zer = jnp.zeros((RB, HH), jnp.float32)

    h, x1 = _embed(z2, emb, conv_lin1[0])
    wfs = [
        _filter(ew3, mlp_w1[i].T, mlp_b1[i].reshape(NF, 1), mlp_w2[i].T,
                mlp_b2[i].reshape(NF, 1))
        for i in range(NI)
    ]
    for i in range(NI):
        aggp = _sc_scatter(x1, wfs[i], src3, dst3, zer)
        h, x1 = _update(h, aggp, conv_lin2_w[i],
                        conv_lin2_b[i].reshape(1, H), lin_w[i],
                        lin_b[i].reshape(1, H), conv_lin1[(i + 1) % NI])

    out_row = _readout(h, bat2, out1_w, out1_b.reshape(1, NGRAPH),
                       out2_w.reshape(1, NGRAPH), out2_b.reshape(1, 1))
    return out_row.reshape(NGRAPH, 1)


# speedup vs baseline: 1.0176x; 1.0176x over previous
import functools
import math

import jax
import jax.numpy as jnp
from jax import lax
from jax.experimental import pallas as pl
from jax.experimental.pallas import tpu as pltpu
from jax.experimental.pallas import tpu_sc as plsc

N = 10000
E = 320000
H = 128
NG = 50
NF = 128
NI = 6
CUT = 10.0
NGRAPH = 64
LOG2 = math.log(2.0)

NC = 2
NS = 16
HH = H // NC
EW = E // NS
CH = 125
NCHUNK = EW // CH
N_PAD = 10240
RPW = N_PAD // NS
RB = 64

TN = 2000
TE = 4000


def _ssp(x):
    return jnp.log(0.5 + 0.5 * jnp.exp(x))


def _embed_body(z_ref, emb_ref, lin1_ref, h_ref, x1_ref):
    zt = z_ref[...]
    ids = lax.broadcasted_iota(jnp.int32, (TN, 85), 1)
    oh = (zt == ids).astype(jnp.float32)
    h = jnp.dot(oh, emb_ref[...], preferred_element_type=jnp.float32)
    h_ref[...] = h
    x1 = jnp.dot(h, lin1_ref[...], preferred_element_type=jnp.float32)
    x1_ref[0] = x1[:, :HH]
    x1_ref[1] = x1[:, HH:]


def _embed(z2, emb, lin1):
    return pl.pallas_call(
        _embed_body,
        grid=(N // TN,),
        in_specs=[
            pl.BlockSpec((TN, 1), lambda i: (i, 0)),
            pl.BlockSpec((85, H), lambda i: (0, 0)),
            pl.BlockSpec((H, H), lambda i: (0, 0)),
        ],
        out_specs=[
            pl.BlockSpec((TN, H), lambda i: (i, 0)),
            pl.BlockSpec((NC, TN, HH), lambda i: (0, i, 0)),
        ],
        out_shape=[
            jax.ShapeDtypeStruct((N, H), jnp.float32),
            jax.ShapeDtypeStruct((NC, N, HH), jnp.float32),
        ],
    )(z2, emb, lin1)


def _filter_body(ew_ref, w1t_ref, b1_ref, w2t_ref, b2_ref, wf_ref):
    d = ew_ref[0]
    off = lax.broadcasted_iota(jnp.int32, (NG, 1), 0).astype(jnp.float32) * (
        CUT / (NG - 1))
    coeff = -0.5 / (CUT / (NG - 1)) ** 2
    diff = d - off
    ea_t = jnp.exp(coeff * diff * diff)
    hid_t = _ssp(jnp.dot(w1t_ref[...], ea_t, preferred_element_type=jnp.float32)
                 + b1_ref[...])
    wf_t = jnp.dot(w2t_ref[...], hid_t, preferred_element_type=jnp.float32) \
        + b2_ref[...]
    u2 = d * d * (jnp.pi / (2.0 * CUT)) ** 2
    cu = 1.0 + u2 * (-0.5 + u2 * (1.0 / 24 + u2 * (-1.0 / 720 + u2 * (
        1.0 / 40320 - u2 / 3628800))))
    wf_ref[...] = (wf_t * (cu * cu)).T


def _filter(ew3, w1t, b1c, w2t, b2c):
    full = lambda s: pl.BlockSpec(s, lambda i: (0, 0))
    return pl.pallas_call(
        _filter_body,
        grid=(E // TE,),
        in_specs=[
            pl.BlockSpec((1, 1, TE), lambda i: (i, 0, 0)),
            full((NF, NG)), full((NF, 1)), full((NF, NF)), full((NF, 1)),
        ],
        out_specs=pl.BlockSpec((TE, NF), lambda i: (i, 0)),
        out_shape=jax.ShapeDtypeStruct((E, NF), jnp.float32),
    )(ew3, w1t, b1c, w2t, b2c)


def _update_body(h_ref, a0_ref, a1_ref, w2c_ref, b2c_ref, lw_ref, lb_ref,
                 lin1n_ref, hn_ref, x1n_ref):
    agg = jnp.concatenate([a0_ref[0], a1_ref[0]], axis=1)
    t = _ssp(jnp.dot(agg, w2c_ref[...], preferred_element_type=jnp.float32)
             + b2c_ref[...])
    x2 = jnp.dot(t, lw_ref[...], preferred_element_type=jnp.float32) + lb_ref[...]
    hn = h_ref[...] + x2
    hn_ref[...] = hn
    x1n = jnp.dot(hn, lin1n_ref[...], preferred_element_type=jnp.float32)
    x1n_ref[0] = x1n[:, :HH]
    x1n_ref[1] = x1n[:, HH:]


def _update(h, aggp, w2c, b2c, lw, lb, lin1n):
    full = lambda s: pl.BlockSpec(s, lambda i: (0, 0))
    return pl.pallas_call(
        _update_body,
        grid=(N // TN,),
        in_specs=[
            pl.BlockSpec((TN, H), lambda i: (i, 0)),
            pl.BlockSpec((1, TN, HH), lambda i: (0, i, 0)),
            pl.BlockSpec((1, TN, HH), lambda i: (1, i, 0)),
            full((NF, H)), full((1, H)), full((H, H)), full((1, H)),
            full((H, H)),
        ],
        out_specs=[
            pl.BlockSpec((TN, H), lambda i: (i, 0)),
            pl.BlockSpec((NC, TN, HH), lambda i: (0, i, 0)),
        ],
        out_shape=[
            jax.ShapeDtypeStruct((N, H), jnp.float32),
            jax.ShapeDtypeStruct((NC, N, HH), jnp.float32),
        ],
    )(h, aggp, aggp, w2c, b2c, lw, lb, lin1n)


def _readout_body(h_ref, bat_ref, o1_ref, o1b_ref, o2_ref, o2b_ref, out_ref):
    i = pl.program_id(0)

    @pl.when(i == 0)
    def _():
        out_ref[...] = jnp.zeros_like(out_ref)

    h2 = _ssp(jnp.dot(h_ref[...], o1_ref[...], preferred_element_type=jnp.float32)
              + o1b_ref[...])
    y = jnp.sum(h2 * o2_ref[...], axis=1, keepdims=True) + o2b_ref[...]
    ids = lax.broadcasted_iota(jnp.int32, (TN, NGRAPH), 1)
    oh = (bat_ref[...] == ids).astype(jnp.float32)
    out_ref[...] += jnp.sum(oh * y, axis=0, keepdims=True)


def _readout(h, bat2, o1, o1b, o2row, o2b):
    full = lambda s: pl.BlockSpec(s, lambda i: (0, 0))
    return pl.pallas_call(
        _readout_body,
        grid=(N // TN,),
        in_specs=[
            pl.BlockSpec((TN, H), lambda i: (i, 0)),
            pl.BlockSpec((TN, 1), lambda i: (i, 0)),
            full((H, NGRAPH)), full((1, NGRAPH)), full((1, NGRAPH)),
            full((1, 1)),
        ],
        out_specs=pl.BlockSpec((1, NGRAPH), lambda i: (0, 0)),
        out_shape=jax.ShapeDtypeStruct((1, NGRAPH), jnp.float32),
    )(h, bat2, o1, o1b, o2row, o2b)


def _sc_body(x1_hbm, wf_hbm, src_hbm, dst_hbm, zer_hbm, out_hbm,
             srcv, dstv, rows4, wfv2, zbuf, aggsh,
             sg0, sg1, sw0, sw1, ss0, ss1, ss2):
    cid = lax.axis_index("c")
    sid = lax.axis_index("s")
    sg = (sg0, sg1)
    sw = (sw0, sw1)
    ss = (ss0, ss1, ss2)

    pltpu.sync_copy(zer_hbm, zbuf)
    for k in range(RPW // RB):
        pltpu.sync_copy(zbuf, aggsh.at[pl.ds(sid * RPW + k * RB, RB)])

    pltpu.sync_copy(src_hbm.at[sid], srcv)
    pltpu.sync_copy(dst_hbm.at[sid], dstv)
    plsc.subcore_barrier()

    ebase = sid * EW

    def issue(j, b, drain=True):
        if drain:
            pltpu.make_async_copy(rows4.at[b % 3], aggsh.at[dstv.at[j]],
                                  ss[b % 3]).wait()
        pltpu.async_copy(x1_hbm.at[cid].at[srcv.at[j]],
                         rows4.at[b % 3], sg[b % 2])
        pltpu.async_copy(
            wf_hbm.at[pl.ds(ebase + j * CH, CH), pl.ds(cid * HH, HH)],
            wfv2.at[b % 2], sw[b % 2])

    def process(j, b, issue_next, drain=True):
        pltpu.make_async_copy(x1_hbm.at[cid].at[srcv.at[j]],
                              rows4.at[b % 3], sg[b % 2]).wait()
        pltpu.make_async_copy(
            wf_hbm.at[pl.ds(ebase + j * CH, CH), pl.ds(cid * HH, HH)],
            wfv2.at[b % 2], sw[b % 2]).wait()

        @plsc.parallel_loop(0, CH, 1, unroll=4)
        def _mul(e):
            for c in range(HH // 16):
                sl = pl.ds(c * 16, 16)
                rows4[b % 3, e, sl] = rows4[b % 3, e, sl] * wfv2[b % 2, e, sl]

        pltpu.async_copy(rows4.at[b % 3], aggsh.at[dstv.at[j]], ss[b % 3],
                         add=True)
        if issue_next:
            issue(j + 2, b + 2, drain)

    issue(0, 0, drain=False)
    issue(1, 1, drain=False)

    process(0, 0, True, drain=False)
    for q in range(1, 6):
        process(q, q, True)

    def rnd(r, carry):
        j = 6 * r
        for q in range(6):
            process(j + q, q, True)
        return carry

    NMAIN = ((NCHUNK - 2) // 6) * 6
    lax.fori_loop(1, NMAIN // 6, rnd, 0)
    for q in range(NMAIN, NCHUNK):
        process(q, q, q + 2 < NCHUNK)
    for b in range(3):
        pltpu.make_async_copy(rows4.at[b], aggsh.at[dstv.at[0]], ss[b]).wait()
    plsc.subcore_barrier()

    for k in range(RPW // RB):
        sl = pl.ds(sid * RPW + k * RB, RB)
        pltpu.sync_copy(aggsh.at[sl], zbuf)
        pltpu.sync_copy(zbuf, out_hbm.at[cid].at[sl])


@functools.cache
def _build_sc_scatter():
    return pl.kernel(
        _sc_body,
        out_type=jax.ShapeDtypeStruct((NC, N_PAD, HH), jnp.float32),
        mesh=plsc.VectorSubcoreMesh(core_axis_name="c", subcore_axis_name="s"),
        scratch_types=[
            pltpu.VMEM((NCHUNK, CH), jnp.int32),
            pltpu.VMEM((NCHUNK, CH), jnp.int32),
            pltpu.VMEM((3, CH, HH), jnp.float32),
            pltpu.VMEM((2, CH, HH), jnp.float32),
            pltpu.VMEM((RB, HH), jnp.float32),
            pltpu.VMEM_SHARED((N_PAD, HH), jnp.float32),
        ] + [pltpu.SemaphoreType.DMA] * 7,
        compiler_params=pltpu.CompilerParams(use_tc_tiling_on_sc=False),
    )


def _sc_scatter(x1, wf4, src3, dst3, zer):
    return _build_sc_scatter()(x1, wf4, src3, dst3, zer)


def kernel(z, edge_index, edge_weight, batch, emb, mlp_w1, mlp_b1, mlp_w2,
           mlp_b2, conv_lin1, conv_lin2_w, conv_lin2_b, lin_w, lin_b,
           out1_w, out1_b, out2_w, out2_b):
    z2 = z.reshape(N, 1).astype(jnp.int32)
    bat2 = batch.reshape(N, 1).astype(jnp.int32)
    ew3 = edge_weight.reshape(E // TE, 1, TE)
    src3 = edge_index[0].reshape(NS, NCHUNK, CH).astype(jnp.int32)
    dst3 = edge_index[1].reshape(NS, NCHUNK, CH).astype(jnp.int32)
    zer = jnp.zeros((RB, HH), jnp.float32)

    h, x1 = _embed(z2, emb, conv_lin1[0])
    wfs = [
        _filter(ew3, mlp_w1[i].T, mlp_b1[i].reshape(NF, 1), mlp_w2[i].T,
                mlp_b2[i].reshape(NF, 1))
        for i in range(NI)
    ]
    for i in range(NI):
        aggp = _sc_scatter(x1, wfs[i], src3, dst3, zer)
        h, x1 = _update(h, aggp, conv_lin2_w[i],
                        conv_lin2_b[i].reshape(1, H), lin_w[i],
                        lin_b[i].reshape(1, H), conv_lin1[(i + 1) % NI])

    out_row = _readout(h, bat2, out1_w, out1_b.reshape(1, NGRAPH),
                       out2_w.reshape(1, NGRAPH), out2_b.reshape(1, 1))
    return out_row.reshape(NGRAPH, 1)
